# baseline (speedup 1.0000x reference)
import functools

import jax
import jax.numpy as jnp
from jax.experimental import pallas as pl

_MC = 1024
_G = 8


def _chamfer_kernel(x1_ref, x2t_ref, d1_ref, d2_ref):
    n = x1_ref.shape[1]
    m = x2t_ref.shape[2]
    ngroups = n // _G
    for mi in range(m // _MC):
        sl = slice(mi * _MC, (mi + 1) * _MC)
        bx = jnp.broadcast_to(x2t_ref[0, 0:1, sl], (_G, _MC))
        by = jnp.broadcast_to(x2t_ref[0, 1:2, sl], (_G, _MC))
        bz = jnp.broadcast_to(x2t_ref[0, 2:3, sl], (_G, _MC))
        n2b = bx * bx + by * by + bz * bz

        def body(g, min2acc, mi=mi, bx=bx, by=by, bz=bz, n2b=n2b):
            x1g = x1_ref[0, pl.ds(g * _G, _G), :]
            ax = x1g[:, 0:1]
            ay = x1g[:, 1:2]
            az = x1g[:, 2:3]
            n12 = (ax * ax + ay * ay + az * az) + n2b
            t = (-2.0 * ax) * bx + (-2.0 * ay) * by + (-2.0 * az) * bz
            d = t + n12
            m1 = jnp.min(d, axis=1, keepdims=True)
            idx = pl.ds(g * _G, _G)
            if mi == 0:
                d1_ref[0, idx, :] = m1
            else:
                d1_ref[0, idx, :] = jnp.minimum(d1_ref[0, idx, :], m1)
            return jnp.minimum(min2acc, d)

        min2acc = jax.lax.fori_loop(
            0, ngroups, body, jnp.full((_G, _MC), jnp.inf, jnp.float32))
        d2_ref[0, 0:1, sl] = jnp.min(min2acc, axis=0, keepdims=True)


@functools.partial(jax.jit, static_argnames=("interpret",))
def kernel(input1, input2, interpret=False):
    xyz1 = input1 if input1.shape[2] == 3 else jnp.transpose(input1, (0, 2, 1))
    xyz2 = input2 if input2.shape[2] == 3 else jnp.transpose(input2, (0, 2, 1))
    B, N, _ = xyz1.shape
    M = xyz2.shape[1]
    x1p = jnp.pad(xyz1, ((0, 0), (0, 0), (0, 5)))
    x2t = jnp.transpose(xyz2, (0, 2, 1))
    d1, d2 = pl.pallas_call(
        _chamfer_kernel,
        grid=(B,),
        in_specs=[
            pl.BlockSpec((1, N, 8), lambda b: (b, 0, 0)),
            pl.BlockSpec((1, 3, M), lambda b: (b, 0, 0)),
        ],
        out_specs=[
            pl.BlockSpec((1, N, 1), lambda b: (b, 0, 0)),
            pl.BlockSpec((1, 1, M), lambda b: (b, 0, 0)),
        ],
        out_shape=[
            jax.ShapeDtypeStruct((B, N, 1), jnp.float32),
            jax.ShapeDtypeStruct((B, 1, M), jnp.float32),
        ],
        interpret=interpret,
    )(x1p, x2t)
    return (d1.reshape(B, N), d2.reshape(B, M))


# speedup vs baseline: 3.6364x; 3.6364x over previous
import functools

import jax
import jax.numpy as jnp
from jax.experimental import pallas as pl

_MC = 1024
_G = 8
_U = 4


def _chamfer_kernel(x1_ref, x2t_ref, d1_ref, d2_ref):
    n = x1_ref.shape[1]
    m = x2t_ref.shape[2]
    ngroups = n // _G
    for mi in range(m // _MC):
        sl = slice(mi * _MC, (mi + 1) * _MC)
        bx = jnp.broadcast_to(x2t_ref[0, 0:1, sl], (_G, _MC))
        by = jnp.broadcast_to(x2t_ref[0, 1:2, sl], (_G, _MC))
        bz = jnp.broadcast_to(x2t_ref[0, 2:3, sl], (_G, _MC))
        n2b = bx * bx + by * by + bz * bz

        def body(i, min2acc, mi=mi, bx=bx, by=by, bz=bz, n2b=n2b):
            ds = []
            for u in range(_U):
                g = i * _U + u
                x1g = x1_ref[0, pl.ds(g * _G, _G), :]
                ax = x1g[:, 0:1]
                ay = x1g[:, 1:2]
                az = x1g[:, 2:3]
                n12 = (ax * ax + ay * ay + az * az) + n2b
                t = (-2.0 * ax) * bx + (-2.0 * ay) * by + (-2.0 * az) * bz
                d = t + n12
                m1 = jnp.min(d, axis=1, keepdims=True)
                idx = pl.ds(g * _G, _G)
                if mi == 0:
                    d1_ref[0, idx, :] = m1
                else:
                    d1_ref[0, idx, :] = jnp.minimum(d1_ref[0, idx, :], m1)
                ds.append(d)
            dmin = jnp.minimum(jnp.minimum(ds[0], ds[1]),
                               jnp.minimum(ds[2], ds[3]))
            return jnp.minimum(min2acc, dmin)

        min2acc = jax.lax.fori_loop(
            0, ngroups // _U, body,
            jnp.full((_G, _MC), jnp.inf, jnp.float32))
        d2_ref[0, 0:1, sl] = jnp.min(min2acc, axis=0, keepdims=True)


@functools.partial(jax.jit, static_argnames=("interpret",))
def kernel(input1, input2, interpret=False):
    xyz1 = input1 if input1.shape[2] == 3 else jnp.transpose(input1, (0, 2, 1))
    xyz2 = input2 if input2.shape[2] == 3 else jnp.transpose(input2, (0, 2, 1))
    B, N, _ = xyz1.shape
    M = xyz2.shape[1]
    x1p = jnp.pad(xyz1, ((0, 0), (0, 0), (0, 5)))
    x2t = jnp.transpose(xyz2, (0, 2, 1))
    d1, d2 = pl.pallas_call(
        _chamfer_kernel,
        grid=(B,),
        in_specs=[
            pl.BlockSpec((1, N, 8), lambda b: (b, 0, 0)),
            pl.BlockSpec((1, 3, M), lambda b: (b, 0, 0)),
        ],
        out_specs=[
            pl.BlockSpec((1, N, 1), lambda b: (b, 0, 0)),
            pl.BlockSpec((1, 1, M), lambda b: (b, 0, 0)),
        ],
        out_shape=[
            jax.ShapeDtypeStruct((B, N, 1), jnp.float32),
            jax.ShapeDtypeStruct((B, 1, M), jnp.float32),
        ],
        interpret=interpret,
    )(x1p, x2t)
    return (d1.reshape(B, N), d2.reshape(B, M))


# speedup vs baseline: 4.4432x; 1.2219x over previous
import functools

import jax
import jax.numpy as jnp
from jax.experimental import pallas as pl

_MC = 1024
_G = 8
_U = 4


def _chamfer_kernel(x1_ref, x2t_ref, d1_ref, d2_ref):
    n = x1_ref.shape[1]
    m = x2t_ref.shape[2]
    ngroups = n // _G
    for mi in range(m // _MC):
        sl = slice(mi * _MC, (mi + 1) * _MC)
        bx = jnp.broadcast_to(x2t_ref[0, 0:1, sl], (_G, _MC))
        by = jnp.broadcast_to(x2t_ref[0, 1:2, sl], (_G, _MC))
        bz = jnp.broadcast_to(x2t_ref[0, 2:3, sl], (_G, _MC))

        def body(i, min2acc, mi=mi, bx=bx, by=by, bz=bz):
            ds = []
            for u in range(_U):
                g = i * _U + u
                x1g = x1_ref[0, pl.ds(g * _G, _G), :]
                ax = x1g[:, 0:1]
                ay = x1g[:, 1:2]
                az = x1g[:, 2:3]
                dx = ax - bx
                dy = ay - by
                dz = az - bz
                d = dx * dx + dy * dy + dz * dz
                m1 = jnp.min(d, axis=1, keepdims=True)
                idx = pl.ds(g * _G, _G)
                if mi == 0:
                    d1_ref[0, idx, :] = m1
                else:
                    d1_ref[0, idx, :] = jnp.minimum(d1_ref[0, idx, :], m1)
                ds.append(d)
            dmin = jnp.minimum(jnp.minimum(ds[0], ds[1]),
                               jnp.minimum(ds[2], ds[3]))
            return jnp.minimum(min2acc, dmin)

        min2acc = jax.lax.fori_loop(
            0, ngroups // _U, body,
            jnp.full((_G, _MC), jnp.inf, jnp.float32))
        d2_ref[0, 0:1, sl] = jnp.min(min2acc, axis=0, keepdims=True)


@functools.partial(jax.jit, static_argnames=("interpret",))
def kernel(input1, input2, interpret=False):
    xyz1 = input1 if input1.shape[2] == 3 else jnp.transpose(input1, (0, 2, 1))
    xyz2 = input2 if input2.shape[2] == 3 else jnp.transpose(input2, (0, 2, 1))
    B, N, _ = xyz1.shape
    M = xyz2.shape[1]
    x1p = jnp.pad(xyz1, ((0, 0), (0, 0), (0, 5)))
    x2t = jnp.transpose(xyz2, (0, 2, 1))
    d1, d2 = pl.pallas_call(
        _chamfer_kernel,
        grid=(B,),
        in_specs=[
            pl.BlockSpec((1, N, 8), lambda b: (b, 0, 0)),
            pl.BlockSpec((1, 3, M), lambda b: (b, 0, 0)),
        ],
        out_specs=[
            pl.BlockSpec((1, N, 1), lambda b: (b, 0, 0)),
            pl.BlockSpec((1, 1, M), lambda b: (b, 0, 0)),
        ],
        out_shape=[
            jax.ShapeDtypeStruct((B, N, 1), jnp.float32),
            jax.ShapeDtypeStruct((B, 1, M), jnp.float32),
        ],
        interpret=interpret,
    )(x1p, x2t)
    return (d1.reshape(B, N), d2.reshape(B, M))


# speedup vs baseline: 11.8475x; 2.6664x over previous
import functools

import jax
import jax.numpy as jnp
from jax.experimental import pallas as pl

_TN = 2048


def _chamfer_kernel(x1_ref, x2t_ref, d1_ref, d2_ref):
    ni = pl.program_id(1)
    x1 = x1_ref[0]
    x2t = x2t_ref[0]
    dx = x1[:, 0:1] - x2t[0:1, :]
    dy = x1[:, 1:2] - x2t[1:2, :]
    dz = x1[:, 2:3] - x2t[2:3, :]
    d = dx * dx + dy * dy + dz * dz
    d1_ref[0] = jnp.min(d, axis=1, keepdims=True)
    m2 = jnp.min(d, axis=0, keepdims=True)

    @pl.when(ni == 0)
    def _init():
        d2_ref[0] = m2

    @pl.when(ni != 0)
    def _acc():
        d2_ref[0] = jnp.minimum(d2_ref[0], m2)


@functools.partial(jax.jit, static_argnames=("interpret",))
def kernel(input1, input2, interpret=False):
    xyz1 = input1 if input1.shape[2] == 3 else jnp.transpose(input1, (0, 2, 1))
    xyz2 = input2 if input2.shape[2] == 3 else jnp.transpose(input2, (0, 2, 1))
    B, N, _ = xyz1.shape
    M = xyz2.shape[1]
    x1p = jnp.pad(xyz1, ((0, 0), (0, 0), (0, 5)))
    x2t = jnp.transpose(xyz2, (0, 2, 1))
    nt = N // _TN
    d1, d2 = pl.pallas_call(
        _chamfer_kernel,
        grid=(B, nt),
        in_specs=[
            pl.BlockSpec((1, _TN, 8), lambda b, i: (b, i, 0)),
            pl.BlockSpec((1, 3, M), lambda b, i: (b, 0, 0)),
        ],
        out_specs=[
            pl.BlockSpec((1, _TN, 1), lambda b, i: (b, i, 0)),
            pl.BlockSpec((1, 1, M), lambda b, i: (b, 0, 0)),
        ],
        out_shape=[
            jax.ShapeDtypeStruct((B, N, 1), jnp.float32),
            jax.ShapeDtypeStruct((B, 1, M), jnp.float32),
        ],
        interpret=interpret,
    )(x1p, x2t)
    return (d1.reshape(B, N), d2.reshape(B, M))
